# baseline (speedup 1.0000x reference)
import functools

import jax
import jax.numpy as jnp
from jax import lax
from jax.experimental import pallas as pl
from jax.experimental.pallas import tpu as pltpu
from jax.experimental.pallas import tpu_sc as plsc

_VOCAB = 100000
_DIM = 128
_BATCH = 4096
_SEQ = 50
_DP = 16
_NW = 32
_B_PER_W = _BATCH // _NW
_TOK_PER_W = _B_PER_W * _SEQ
_CHUNK = 128
_NCHUNK = _TOK_PER_W // _CHUNK
_ROWS_BLK = 25000


def _proj_body(emb_ref, w_ref, out_ref):
    w = jnp.concatenate(
        [w_ref[...] * (1.0 / _SEQ), jnp.zeros((_DIM, 126), jnp.float32)], axis=1
    )
    out_ref[...] = jnp.dot(emb_ref[...], w, preferred_element_type=jnp.float32)


def _project(emb_table, W_out):
    return pl.pallas_call(
        _proj_body,
        grid=(_VOCAB // _ROWS_BLK,),
        in_specs=[
            pl.BlockSpec((_ROWS_BLK, _DIM), lambda i: (i, 0)),
            pl.BlockSpec((_DIM, 2), lambda i: (0, 0)),
        ],
        out_specs=pl.BlockSpec((_ROWS_BLK, 128), lambda i: (i, 0)),
        out_shape=jax.ShapeDtypeStruct((_VOCAB, 128), jnp.float32),
    )(emb_table, W_out)


def _pool_body(p_hbm, idx_hbm, out_hbm, idx_v, rows_v, acc_v, sem1, sem2):
    wid = lax.axis_index("s") * 2 + lax.axis_index("c")
    half = _B_PER_W // 2

    pltpu.sync_copy(idx_hbm.at[pl.ds(wid * _B_PER_W, _B_PER_W)], idx_v)

    def _fire(sem):
        def go(r, carry):
            pltpu.async_copy(
                p_hbm.at[idx_v.at[r]],
                rows_v.at[pl.ds(r * _SEQ, _SEQ)],
                sem,
            )
            return carry

        return go

    lax.fori_loop(0, half, _fire(sem1), 0)
    lax.fori_loop(half, _B_PER_W, _fire(sem2), 0)

    def _acc(b, carry):
        base = b * _SEQ
        acc = rows_v[base, :]
        for l in range(1, _SEQ):
            acc = acc + rows_v[base + l, :]
        acc_v[b, :] = acc
        return carry

    hrows = half * _SEQ
    pltpu.make_async_copy(
        p_hbm.at[pl.ds(0, hrows)], rows_v.at[pl.ds(0, hrows)], sem1
    ).wait()
    lax.fori_loop(0, half, _acc, 0)
    pltpu.make_async_copy(
        p_hbm.at[pl.ds(0, hrows)], rows_v.at[pl.ds(hrows, hrows)], sem2
    ).wait()
    lax.fori_loop(half, _B_PER_W, _acc, 0)

    pltpu.sync_copy(acc_v, out_hbm.at[pl.ds(wid * _B_PER_W, _B_PER_W)])


def _pool(p, idx2):
    mesh = plsc.VectorSubcoreMesh(
        core_axis_name="c", subcore_axis_name="s", num_cores=2, num_subcores=16
    )
    fn = pl.kernel(
        _pool_body,
        out_type=jax.ShapeDtypeStruct((_BATCH, _DP), jnp.float32),
        mesh=mesh,
        scratch_types=[
            pltpu.VMEM((_B_PER_W, _SEQ), jnp.int32),
            pltpu.VMEM((_TOK_PER_W, _DP), jnp.float32),
            pltpu.VMEM((_B_PER_W, _DP), jnp.float32),
            pltpu.SemaphoreType.DMA,
            pltpu.SemaphoreType.DMA,
        ],
        compiler_params=pltpu.CompilerParams(use_tc_tiling_on_sc=False),
    )
    return fn(p, idx2)


def _lsm_body(x_ref, b_ref, o_ref):
    x = x_ref[:, 0:2] + b_ref[...]
    x0 = x[:, 0:1]
    x1 = x[:, 1:2]
    m = jnp.maximum(x0, x1)
    lse = jnp.log(jnp.exp(x0 - m) + jnp.exp(x1 - m)) + m
    o_ref[...] = x - lse


def _log_softmax2(pooled, b_out):
    return pl.pallas_call(
        _lsm_body,
        out_shape=jax.ShapeDtypeStruct((_BATCH, 2), jnp.float32),
    )(pooled, b_out.reshape(1, 2))


def kernel(input, emb_table, W_out, b_out):
    p = _project(emb_table, W_out).reshape(_VOCAB * 8, _DP)
    pooled = _pool(p, input.astype(jnp.int32) * 8)
    return _log_softmax2(pooled, b_out)


# speedup vs baseline: 1.0691x; 1.0691x over previous
import functools

import jax
import jax.numpy as jnp
from jax import lax
from jax.experimental import pallas as pl
from jax.experimental.pallas import tpu as pltpu
from jax.experimental.pallas import tpu_sc as plsc

_VOCAB = 100000
_DIM = 128
_BATCH = 4096
_SEQ = 50
_DP = 16
_NW = 32
_B_PER_W = _BATCH // _NW
_TOK_PER_W = _B_PER_W * _SEQ
_CHUNK = 128
_NCHUNK = _TOK_PER_W // _CHUNK
_ROWS_BLK = 20000


def _proj_body(emb_ref, w_ref, out_ref):
    w = jnp.concatenate(
        [w_ref[...] * (1.0 / _SEQ), jnp.zeros((_DIM, 126), jnp.float32)], axis=1
    )
    out_ref[...] = jnp.dot(emb_ref[...], w, preferred_element_type=jnp.float32)


def _project(emb_table, W_out):
    return pl.pallas_call(
        _proj_body,
        grid=(_VOCAB // _ROWS_BLK,),
        in_specs=[
            pl.BlockSpec((_ROWS_BLK, _DIM), lambda i: (i, 0)),
            pl.BlockSpec((_DIM, 2), lambda i: (0, 0)),
        ],
        out_specs=pl.BlockSpec((_ROWS_BLK, 128), lambda i: (i, 0)),
        out_shape=jax.ShapeDtypeStruct((_VOCAB, 128), jnp.float32),
    )(emb_table, W_out)


def _pool_body(p_hbm, idx_hbm, out_hbm, idx_v, rows_v, acc_v, sem1, sem2):
    wid = lax.axis_index("s") * 2 + lax.axis_index("c")
    half = _B_PER_W // 2

    pltpu.sync_copy(idx_hbm.at[pl.ds(wid * _B_PER_W, _B_PER_W)], idx_v)

    def _fire(sem):
        def go(r, carry):
            pltpu.async_copy(
                p_hbm.at[idx_v.at[r]],
                rows_v.at[pl.ds(r * _SEQ, _SEQ)],
                sem,
            )
            return carry

        return go

    lax.fori_loop(0, half, _fire(sem1), 0)
    lax.fori_loop(half, _B_PER_W, _fire(sem2), 0)

    def _acc(b, carry):
        base = b * _SEQ
        part = [rows_v[base + k, :] for k in range(4)]
        for l in range(4, _SEQ - 2, 4):
            for k in range(4):
                part[k] = part[k] + rows_v[base + l + k, :]
        part[0] = part[0] + rows_v[base + _SEQ - 2, :]
        part[1] = part[1] + rows_v[base + _SEQ - 1, :]
        acc_v[b, :] = (part[0] + part[1]) + (part[2] + part[3])
        return carry

    hrows = half * _SEQ
    pltpu.make_async_copy(
        p_hbm.at[pl.ds(0, hrows)], rows_v.at[pl.ds(0, hrows)], sem1
    ).wait()
    lax.fori_loop(0, half, _acc, 0)
    pltpu.make_async_copy(
        p_hbm.at[pl.ds(0, hrows)], rows_v.at[pl.ds(hrows, hrows)], sem2
    ).wait()
    lax.fori_loop(half, _B_PER_W, _acc, 0)

    pltpu.sync_copy(acc_v, out_hbm.at[pl.ds(wid * _B_PER_W, _B_PER_W)])


def _pool(p, idx2):
    mesh = plsc.VectorSubcoreMesh(
        core_axis_name="c", subcore_axis_name="s", num_cores=2, num_subcores=16
    )
    fn = pl.kernel(
        _pool_body,
        out_type=jax.ShapeDtypeStruct((_BATCH, _DP), jnp.float32),
        mesh=mesh,
        scratch_types=[
            pltpu.VMEM((_B_PER_W, _SEQ), jnp.int32),
            pltpu.VMEM((_TOK_PER_W, _DP), jnp.float32),
            pltpu.VMEM((_B_PER_W, _DP), jnp.float32),
            pltpu.SemaphoreType.DMA,
            pltpu.SemaphoreType.DMA,
        ],
        compiler_params=pltpu.CompilerParams(use_tc_tiling_on_sc=False),
    )
    return fn(p, idx2)


def _lsm_body(x_ref, b_ref, o_ref):
    x = x_ref[:, 0:2] + b_ref[...]
    x0 = x[:, 0:1]
    x1 = x[:, 1:2]
    m = jnp.maximum(x0, x1)
    lse = jnp.log(jnp.exp(x0 - m) + jnp.exp(x1 - m)) + m
    o_ref[...] = x - lse


def _log_softmax2(pooled, b_out):
    return pl.pallas_call(
        _lsm_body,
        out_shape=jax.ShapeDtypeStruct((_BATCH, 2), jnp.float32),
    )(pooled, b_out.reshape(1, 2))


def kernel(input, emb_table, W_out, b_out):
    p = _project(emb_table, W_out).reshape(_VOCAB * 8, _DP)
    pooled = _pool(p, input.astype(jnp.int32) * 8)
    return _log_softmax2(pooled, b_out)
